# baseline (speedup 1.0000x reference)
import functools

import jax
import jax.numpy as jnp
from jax import lax
from jax.experimental import pallas as pl
from jax.experimental.pallas import tpu as pltpu
from jax.experimental.pallas import tpu_sc as plsc

_G = 128
_NC = 2
_NS = 16
_NW = _NC * _NS
_LANES = 16


@functools.lru_cache(maxsize=None)
def _make_gather2x(n_groups: int, n_rows: int, d: int):
    n_full = n_rows // _G
    rem = n_rows - n_full * _G
    t_max = -(-n_groups // _NW)
    t_pad = -(-(t_max + 8) // 8) * 8
    n_groups_pad = max(
        ((w * n_groups) // _NW // 8) * 8 + t_pad for w in range(_NW)
    )
    mesh = plsc.VectorSubcoreMesh(
        core_axis_name="c", subcore_axis_name="s",
        num_cores=_NC, num_subcores=_NS,
    )

    n_buf = 4
    t_full = (n_groups // _NW) // n_buf * n_buf

    @functools.partial(
        pl.kernel,
        out_type=jax.ShapeDtypeStruct((n_rows, d), jnp.float32),
        mesh=mesh,
        scratch_types=[
            pltpu.VMEM((t_pad, _G), jnp.int32),
            pltpu.VMEM((n_buf, _G, d), jnp.float32),
            [pltpu.SemaphoreType.DMA] * n_buf,
            [pltpu.SemaphoreType.DMA] * n_buf,
        ],
    )
    def gather2x(table_hbm, idx_hbm, out_hbm, idx_v, rows_v, sem_g, sem_o):
        wid = lax.axis_index("s") * _NC + lax.axis_index("c")
        g0 = (wid * n_groups) // _NW
        cnt = ((wid + 1) * n_groups) // _NW - g0
        a0 = pl.multiple_of((g0 // 8) * 8, 8)
        off = g0 - a0
        pltpu.sync_copy(idx_hbm.at[pl.ds(a0, t_pad)], idx_v)

        def start(t, buf):
            pltpu.async_copy(
                table_hbm.at[idx_v.at[t + off]], rows_v.at[buf], sem_g[buf]
            )

        def wait_gather(buf):
            pltpu.make_async_copy(
                table_hbm.at[pl.ds(0, _G)], rows_v.at[buf], sem_g[buf]
            ).wait()

        def scale(buf):
            @pl.loop(0, _G, unroll=8)
            def _(r):
                for k in range(d // _LANES):
                    sl = pl.ds(k * _LANES, _LANES)
                    v = rows_v[buf, r, sl]
                    rows_v[buf, r, sl] = v + v

        def _write_parts(buf, g, go):
            @pl.when(g < n_full)
            def _():
                go(rows_v.at[buf], out_hbm.at[pl.ds(g * _G, _G)], sem_o[buf])

            if rem:
                @pl.when(g == n_full)
                def _():
                    go(
                        rows_v.at[buf, pl.ds(0, rem)],
                        out_hbm.at[pl.ds(n_full * _G, rem)],
                        sem_o[buf],
                    )

        def flush_async(buf, g):
            _write_parts(buf, g, pltpu.async_copy)

        def drain_write(buf, g):
            _write_parts(
                buf, g, lambda s, dd, sm: pltpu.make_async_copy(s, dd, sm).wait()
            )

        def step(t, buf):
            wait_gather(buf)
            scale(buf)
            flush_async(buf, g0 + t)
            nxt = (buf + 2) % n_buf

            @pl.when(t >= 2)
            def _():
                drain_write(nxt, g0 + t - 2)

            @pl.when(t + 2 < cnt)
            def _():
                start(t + 2, nxt)

        start(0, 0)
        start(1, 1)

        @pl.loop(0, t_full // n_buf)
        def _(p):
            for i in range(n_buf):
                step(p * n_buf + i, i)

        for i in range(n_buf):
            t = t_full + i

            @pl.when(t < cnt)
            def _():
                step(t, t % n_buf)

        for dt in (2, 1):
            for b in range(n_buf):
                @pl.when((cnt - dt) % n_buf == b)
                def _():
                    drain_write(b, g0 + cnt - dt)

    return gather2x, n_groups_pad


def kernel(memory, source_nodes, timestamps, n_layers, time_w, time_b):
    del timestamps, n_layers, time_w, time_b
    n_rows = source_nodes.shape[0]
    d = memory.shape[1]
    idx = source_nodes.astype(jnp.int32)
    n_groups = (n_rows + _G - 1) // _G
    fn, n_groups_pad = _make_gather2x(n_groups, n_rows, d)
    idx2d = jnp.pad(idx, (0, n_groups_pad * _G - n_rows)).reshape(
        n_groups_pad, _G
    )
    return fn(memory, idx2d)
